# baseline (speedup 1.0000x reference)
import dataclasses
import functools
import math

import jax
import jax.numpy as jnp
from jax import lax
from jax.experimental import pallas as pl
from jax.experimental.pallas import tpu as pltpu
from jax.experimental.pallas import tpu_sc as plsc

N = 10000
E = 320000
D = 128
NC = 2
NS = 16
NW = NC * NS
CHUNK = 64
NCH = 160
SCHUNK = 128
SNCH = 80
PNCH = 40
NPH = 2
NB = 2
E_PAD = NW * NCH * CHUNK
PADT = E_PAD - E
N_ACC = 10112
TPW = N_ACC // NS
N_DEG = 10240
TPD = N_DEG // NS
RB = 2000

ALPHA = 0.1
THETA1 = math.log(2.0)
THETA2 = math.log(1.5)

_MESH = plsc.VectorSubcoreMesh(core_axis_name="c", subcore_axis_name="s")

_CP = pltpu.CompilerParams()
if "needs_layout_passes" in pltpu.CompilerParams.__dataclass_fields__:
    _CP = dataclasses.replace(_CP, needs_layout_passes=False)


def _deg_partials(colh):
    @functools.partial(
        pl.kernel,
        out_type=jax.ShapeDtypeStruct((NC, N_DEG, D), jnp.float32),
        mesh=_MESH,
        compiler_params=_CP,
        scratch_types=[
            pltpu.VMEM((NCH, CHUNK), jnp.int32),
            pltpu.VMEM((N_DEG,), jnp.float32),
            pltpu.VMEM((NS, TPD), jnp.float32),
            pltpu.VMEM((TPD,), jnp.float32),
            pltpu.VMEM((TPD // 2, D), jnp.float32),
            pltpu.VMEM_SHARED((NS, N_DEG), jnp.float32),
        ],
    )
    def k(colh_hbm, out_hbm, colv, hist, redbuf, tot, outbuf, shist):
        c = lax.axis_index("c")
        s = lax.axis_index("s")
        w = c * NS + s
        pltpu.sync_copy(colh_hbm.at[w], colv)

        @pl.loop(0, N_DEG // 16)
        def _(i):
            hist[pl.ds(i * 16, 16)] = jnp.zeros((16,), jnp.float32)

        ones = jnp.ones((16,), jnp.float32)

        @pl.loop(0, NCH)
        def _(j):
            for k in range(CHUNK // 16):
                idx = colv[j, pl.ds(k * 16, 16)]
                plsc.addupdate_scatter(hist, [idx], ones)

        pltpu.sync_copy(hist, shist.at[s])
        plsc.subcore_barrier()
        for t in range(NS):
            pltpu.sync_copy(shist.at[t].at[pl.ds(s * TPD, TPD)],
                            redbuf.at[t])

        @pl.loop(0, TPD // 16)
        def _(i):
            v = redbuf[0, pl.ds(i * 16, 16)]
            for t in range(1, NS):
                v = v + redbuf[t, pl.ds(i * 16, 16)]
            tot[pl.ds(i * 16, 16)] = v

        for h in range(2):
            @pl.loop(0, TPD // 32)
            def _(i):
                v = tot[pl.ds(h * (TPD // 2) + i * 16, 16)]
                for k in range(16):
                    outbuf[i * 16 + k, pl.ds(0, 16)] = jnp.full((16,), v[k])

            pltpu.sync_copy(
                outbuf,
                out_hbm.at[c].at[pl.ds(s * TPD + h * (TPD // 2), TPD // 2)])

    return k(colh)


def _spmm_partials(g, colg, rows, zeros128):
    @functools.partial(
        pl.kernel,
        out_type=jax.ShapeDtypeStruct((NC, N_ACC, D), jnp.float32),
        mesh=_MESH,
        scratch_types=[
            pltpu.VMEM((PNCH, SCHUNK), jnp.int32),
            pltpu.VMEM((PNCH, SCHUNK), jnp.int32),
            pltpu.VMEM((SCHUNK, D), jnp.float32),
            pltpu.VMEM((SCHUNK, D), jnp.float32),
            pltpu.VMEM_SHARED((N_ACC, D), jnp.float32),
            pltpu.SemaphoreType.DMA,
            pltpu.SemaphoreType.DMA,
            pltpu.SemaphoreType.DMA,
            pltpu.SemaphoreType.DMA,
        ],
    )
    def k(g_hbm, colg_hbm, rows_hbm, zeros_hbm, out_hbm, colv, rowv,
          d0, d1, acc, gs0, gs1, ss0, ss1):
        c = lax.axis_index("c")
        s = lax.axis_index("s")
        w = c * NS + s
        pltpu.sync_copy(zeros_hbm, acc.at[pl.ds(s * TPW, TPW)])

        bufs = (d0, d1)
        gsems = (gs0, gs1)
        ssems = (ss0, ss1)

        for ph in range(NPH):
            pltpu.sync_copy(colg_hbm.at[w].at[pl.ds(ph * PNCH, PNCH)], colv)
            pltpu.sync_copy(rows_hbm.at[w].at[pl.ds(ph * PNCH, PNCH)], rowv)
            if ph == 0:
                plsc.subcore_barrier()
            for b in range(NB):
                pltpu.make_async_copy(
                    g_hbm.at[colv.at[b]], bufs[b], gsems[b]).start()

            @pl.loop(0, PNCH, step=NB)
            def _(j):
                for b in range(NB):
                    pltpu.make_async_copy(
                        g_hbm.at[colv.at[j + b]], bufs[b], gsems[b]).wait()
                    pltpu.make_async_copy(
                        bufs[b], acc.at[rowv.at[j + b]],
                        ssems[b]).start(add=True)
                for b in range(NB):
                    pltpu.make_async_copy(
                        bufs[b], acc.at[rowv.at[j + b]], ssems[b]).wait()

                    @pl.when(j + NB + b < PNCH)
                    def _():
                        pltpu.make_async_copy(
                            g_hbm.at[colv.at[j + NB + b]], bufs[b],
                            gsems[b]).start()

        plsc.subcore_barrier()
        pltpu.sync_copy(acc.at[pl.ds(s * TPW, TPW)],
                        out_hbm.at[c].at[pl.ds(s * TPW, TPW)])

    return k(g, colg, rows, zeros128)


def _fc0(x, fc0_w, fc0_b):
    def body(x_ref, w_ref, b_ref, o_ref):
        o_ref[...] = jnp.maximum(
            jnp.dot(x_ref[...], w_ref[...],
                    preferred_element_type=jnp.float32,
                    precision=lax.Precision.HIGHEST) + b_ref[...], 0.0)

    return pl.pallas_call(
        body,
        grid=(N // RB,),
        in_specs=[
            pl.BlockSpec((RB, D), lambda i: (i, 0)),
            pl.BlockSpec((D, D), lambda i: (0, 0)),
            pl.BlockSpec((1, D), lambda i: (0, 0)),
        ],
        out_specs=pl.BlockSpec((RB, D), lambda i: (i, 0)),
        out_shape=jax.ShapeDtypeStruct((N, D), jnp.float32),
    )(x, fc0_w, fc0_b.reshape(1, D))


def _prep(degp, h):
    def body(degp_ref, h_ref, dis_ref, g_ref):
        d = degp_ref[...]
        ids = (pl.program_id(0) * RB
               + lax.broadcasted_iota(jnp.int32, (RB, 1), 0))
        corr = jnp.where(ids < PADT, 1.0, 0.0)
        deg = d[0][:, :1] + d[1][:, :1] + 1.0 - corr
        dis = lax.rsqrt(deg)
        dis_b = jnp.broadcast_to(dis, (RB, D))
        dis_ref[...] = dis_b
        g_ref[...] = dis_b * h_ref[...]

    return pl.pallas_call(
        body,
        grid=(N // RB,),
        in_specs=[
            pl.BlockSpec((NC, RB, D), lambda i: (0, i, 0)),
            pl.BlockSpec((RB, D), lambda i: (i, 0)),
        ],
        out_specs=[
            pl.BlockSpec((RB, D), lambda i: (i, 0)),
            pl.BlockSpec((RB, D), lambda i: (i, 0)),
        ],
        out_shape=[
            jax.ShapeDtypeStruct((N, D), jnp.float32),
            jax.ShapeDtypeStruct((N, D), jnp.float32),
        ],
    )(degp, h)


def _layer(sp, g, h0, dis, w, theta):
    def body(sp_ref, g_ref, h0_ref, dis_ref, w_ref, o_ref):
        s = sp_ref[...]
        dis_b = dis_ref[...]
        hi = dis_b * (s[0] + s[1] + g_ref[...])
        sup = (1.0 - ALPHA) * hi + ALPHA * h0_ref[...]
        hn = jnp.maximum(
            theta * jnp.dot(sup, w_ref[...],
                            preferred_element_type=jnp.float32,
                            precision=lax.Precision.HIGHEST)
            + (1.0 - theta) * sup, 0.0)
        o_ref[...] = dis_b * hn

    return pl.pallas_call(
        body,
        grid=(N // RB,),
        in_specs=[
            pl.BlockSpec((NC, RB, D), lambda i: (0, i, 0)),
            pl.BlockSpec((RB, D), lambda i: (i, 0)),
            pl.BlockSpec((RB, D), lambda i: (i, 0)),
            pl.BlockSpec((RB, D), lambda i: (i, 0)),
            pl.BlockSpec((D, D), lambda i: (0, 0)),
        ],
        out_specs=pl.BlockSpec((RB, D), lambda i: (i, 0)),
        out_shape=jax.ShapeDtypeStruct((N, D), jnp.float32),
    )(sp, g, h0, dis, w)


def _final(sp, g1, h0, dis, w2, fc1_w, fc1_b):
    def body(sp_ref, g_ref, h0_ref, dis_ref, w_ref, fw_ref, fb_ref, o_ref):
        s = sp_ref[...]
        hi = dis_ref[...] * (s[0] + s[1] + g_ref[...])
        sup = (1.0 - ALPHA) * hi + ALPHA * h0_ref[...]
        h2 = jnp.maximum(
            THETA2 * jnp.dot(sup, w_ref[...],
                             preferred_element_type=jnp.float32,
                             precision=lax.Precision.HIGHEST)
            + (1.0 - THETA2) * sup, 0.0)
        o = jnp.dot(h2, fw_ref[...],
                    preferred_element_type=jnp.float32,
                    precision=lax.Precision.HIGHEST) + fb_ref[...]
        m = jnp.max(o, axis=1, keepdims=True)
        o_ref[...] = o - m - jnp.log(
            jnp.sum(jnp.exp(o - m), axis=1, keepdims=True))

    return pl.pallas_call(
        body,
        grid=(N // RB,),
        in_specs=[
            pl.BlockSpec((NC, RB, D), lambda i: (0, i, 0)),
            pl.BlockSpec((RB, D), lambda i: (i, 0)),
            pl.BlockSpec((RB, D), lambda i: (i, 0)),
            pl.BlockSpec((RB, D), lambda i: (i, 0)),
            pl.BlockSpec((D, D), lambda i: (0, 0)),
            pl.BlockSpec((D, D), lambda i: (0, 0)),
            pl.BlockSpec((1, D), lambda i: (0, 0)),
        ],
        out_specs=pl.BlockSpec((RB, D), lambda i: (i, 0)),
        out_shape=jax.ShapeDtypeStruct((N, D), jnp.float32),
    )(sp, g1, h0, dis, w2, fc1_w, fc1_b.reshape(1, D))


def kernel(x, edge_index, fc0_w, fc0_b, w1, w2, fc1_w, fc1_b):
    ar = jnp.arange(PADT, dtype=jnp.int32)
    pad_rc = jnp.stack([N + ar % (N_ACC - N), ar])
    eip = jnp.concatenate([edge_index, pad_rc], axis=1)
    rows = eip[0].reshape(NW, SNCH, SCHUNK)
    cols = eip[1].reshape(NW, SNCH, SCHUNK)
    colsd = eip[1].reshape(NW, NCH, CHUNK)
    zeros128 = jnp.zeros((TPW, D), jnp.float32)

    degp = _deg_partials(colsd)
    h0 = _fc0(x, fc0_w, fc0_b)
    dis, g0 = _prep(degp, h0)
    sp1 = _spmm_partials(g0, cols, rows, zeros128)
    g1 = _layer(sp1, g0, h0, dis, w1, THETA1)
    sp2 = _spmm_partials(g1, cols, rows, zeros128)
    return _final(sp2, g1, h0, dis, w2, fc1_w, fc1_b)


# speedup vs baseline: 1.1044x; 1.1044x over previous
import dataclasses
import functools
import math

import jax
import jax.numpy as jnp
from jax import lax
from jax.experimental import pallas as pl
from jax.experimental.pallas import tpu as pltpu
from jax.experimental.pallas import tpu_sc as plsc

N = 10000
E = 320000
D = 128
NC = 2
NS = 16
NW = NC * NS
CHUNK = 64
NCH = 160
SCHUNK = 32
SNCH = 320
PNCH = 40
NPH = 8
NB = 8
E_PAD = NW * NCH * CHUNK
PADT = E_PAD - E
N_ACC = 10112
TPW = N_ACC // NS
N_DEG = 10240
TPD = N_DEG // NS
RB = 2000

ALPHA = 0.1
THETA1 = math.log(2.0)
THETA2 = math.log(1.5)

_MESH = plsc.VectorSubcoreMesh(core_axis_name="c", subcore_axis_name="s")

_CP = pltpu.CompilerParams()
if "needs_layout_passes" in pltpu.CompilerParams.__dataclass_fields__:
    _CP = dataclasses.replace(_CP, needs_layout_passes=False)


def _deg_partials(colh):
    @functools.partial(
        pl.kernel,
        out_type=jax.ShapeDtypeStruct((NC, N_DEG, D), jnp.float32),
        mesh=_MESH,
        compiler_params=_CP,
        scratch_types=[
            pltpu.VMEM((NCH, CHUNK), jnp.int32),
            pltpu.VMEM((N_DEG,), jnp.float32),
            pltpu.VMEM((NS, TPD), jnp.float32),
            pltpu.VMEM((TPD,), jnp.float32),
            pltpu.VMEM((TPD // 2, D), jnp.float32),
            pltpu.VMEM_SHARED((NS, N_DEG), jnp.float32),
        ],
    )
    def k(colh_hbm, out_hbm, colv, hist, redbuf, tot, outbuf, shist):
        c = lax.axis_index("c")
        s = lax.axis_index("s")
        w = c * NS + s
        pltpu.sync_copy(colh_hbm.at[w], colv)

        @pl.loop(0, N_DEG // 16)
        def _(i):
            hist[pl.ds(i * 16, 16)] = jnp.zeros((16,), jnp.float32)

        ones = jnp.ones((16,), jnp.float32)

        @pl.loop(0, NCH)
        def _(j):
            for k in range(CHUNK // 16):
                idx = colv[j, pl.ds(k * 16, 16)]
                plsc.addupdate_scatter(hist, [idx], ones)

        pltpu.sync_copy(hist, shist.at[s])
        plsc.subcore_barrier()
        for t in range(NS):
            pltpu.sync_copy(shist.at[t].at[pl.ds(s * TPD, TPD)],
                            redbuf.at[t])

        @pl.loop(0, TPD // 16)
        def _(i):
            v = redbuf[0, pl.ds(i * 16, 16)]
            for t in range(1, NS):
                v = v + redbuf[t, pl.ds(i * 16, 16)]
            tot[pl.ds(i * 16, 16)] = v

        for h in range(2):
            @pl.loop(0, TPD // 32)
            def _(i):
                v = tot[pl.ds(h * (TPD // 2) + i * 16, 16)]
                for k in range(16):
                    outbuf[i * 16 + k, pl.ds(0, 16)] = jnp.full((16,), v[k])

            pltpu.sync_copy(
                outbuf,
                out_hbm.at[c].at[pl.ds(s * TPD + h * (TPD // 2), TPD // 2)])

    return k(colh)


def _spmm_partials(g, colg, rows, zeros128):
    @functools.partial(
        pl.kernel,
        out_type=jax.ShapeDtypeStruct((NC, N_ACC, D), jnp.float32),
        mesh=_MESH,
        scratch_types=[
            pltpu.VMEM((PNCH, SCHUNK), jnp.int32),
            pltpu.VMEM((PNCH, SCHUNK), jnp.int32),
            pltpu.VMEM((SCHUNK, D), jnp.float32),
            pltpu.VMEM((SCHUNK, D), jnp.float32),
            pltpu.VMEM((SCHUNK, D), jnp.float32),
            pltpu.VMEM((SCHUNK, D), jnp.float32),
            pltpu.VMEM((SCHUNK, D), jnp.float32),
            pltpu.VMEM((SCHUNK, D), jnp.float32),
            pltpu.VMEM((SCHUNK, D), jnp.float32),
            pltpu.VMEM((SCHUNK, D), jnp.float32),
            pltpu.VMEM_SHARED((N_ACC, D), jnp.float32),
            pltpu.SemaphoreType.DMA,
            pltpu.SemaphoreType.DMA,
            pltpu.SemaphoreType.DMA,
            pltpu.SemaphoreType.DMA,
            pltpu.SemaphoreType.DMA,
            pltpu.SemaphoreType.DMA,
            pltpu.SemaphoreType.DMA,
            pltpu.SemaphoreType.DMA,
            pltpu.SemaphoreType.DMA,
            pltpu.SemaphoreType.DMA,
            pltpu.SemaphoreType.DMA,
            pltpu.SemaphoreType.DMA,
            pltpu.SemaphoreType.DMA,
            pltpu.SemaphoreType.DMA,
            pltpu.SemaphoreType.DMA,
            pltpu.SemaphoreType.DMA,
        ],
    )
    def k(g_hbm, colg_hbm, rows_hbm, zeros_hbm, out_hbm, colv, rowv,
          d0, d1, d2, d3, d4, d5, d6, d7, acc,
          gs0, gs1, gs2, gs3, gs4, gs5, gs6, gs7,
          ss0, ss1, ss2, ss3, ss4, ss5, ss6, ss7):
        c = lax.axis_index("c")
        s = lax.axis_index("s")
        w = c * NS + s
        pltpu.sync_copy(zeros_hbm, acc.at[pl.ds(s * TPW, TPW)])

        bufs = (d0, d1, d2, d3, d4, d5, d6, d7)
        gsems = (gs0, gs1, gs2, gs3, gs4, gs5, gs6, gs7)
        ssems = (ss0, ss1, ss2, ss3, ss4, ss5, ss6, ss7)

        for ph in range(NPH):
            pltpu.sync_copy(colg_hbm.at[w].at[pl.ds(ph * PNCH, PNCH)], colv)
            pltpu.sync_copy(rows_hbm.at[w].at[pl.ds(ph * PNCH, PNCH)], rowv)
            if ph == 0:
                plsc.subcore_barrier()
            for b in range(NB):
                pltpu.make_async_copy(
                    g_hbm.at[colv.at[b]], bufs[b], gsems[b]).start()

            @pl.loop(0, PNCH, step=NB)
            def _(j):
                for b in range(NB):
                    pltpu.make_async_copy(
                        g_hbm.at[colv.at[j + b]], bufs[b], gsems[b]).wait()
                    pltpu.make_async_copy(
                        bufs[b], acc.at[rowv.at[j + b]],
                        ssems[b]).start(add=True)
                for b in range(NB):
                    pltpu.make_async_copy(
                        bufs[b], acc.at[rowv.at[j + b]], ssems[b]).wait()

                    @pl.when(j + NB + b < PNCH)
                    def _():
                        pltpu.make_async_copy(
                            g_hbm.at[colv.at[j + NB + b]], bufs[b],
                            gsems[b]).start()

        plsc.subcore_barrier()
        pltpu.sync_copy(acc.at[pl.ds(s * TPW, TPW)],
                        out_hbm.at[c].at[pl.ds(s * TPW, TPW)])

    return k(g, colg, rows, zeros128)


def _fc0(x, fc0_w, fc0_b):
    def body(x_ref, w_ref, b_ref, o_ref):
        o_ref[...] = jnp.maximum(
            jnp.dot(x_ref[...], w_ref[...],
                    preferred_element_type=jnp.float32,
                    precision=lax.Precision.HIGHEST) + b_ref[...], 0.0)

    return pl.pallas_call(
        body,
        grid=(N // RB,),
        in_specs=[
            pl.BlockSpec((RB, D), lambda i: (i, 0)),
            pl.BlockSpec((D, D), lambda i: (0, 0)),
            pl.BlockSpec((1, D), lambda i: (0, 0)),
        ],
        out_specs=pl.BlockSpec((RB, D), lambda i: (i, 0)),
        out_shape=jax.ShapeDtypeStruct((N, D), jnp.float32),
    )(x, fc0_w, fc0_b.reshape(1, D))


def _prep(degp, h):
    def body(degp_ref, h_ref, dis_ref, g_ref):
        d = degp_ref[...]
        ids = (pl.program_id(0) * RB
               + lax.broadcasted_iota(jnp.int32, (RB, 1), 0))
        corr = jnp.where(ids < PADT, 1.0, 0.0)
        deg = d[0][:, :1] + d[1][:, :1] + 1.0 - corr
        dis = lax.rsqrt(deg)
        dis_b = jnp.broadcast_to(dis, (RB, D))
        dis_ref[...] = dis_b
        g_ref[...] = dis_b * h_ref[...]

    return pl.pallas_call(
        body,
        grid=(N // RB,),
        in_specs=[
            pl.BlockSpec((NC, RB, D), lambda i: (0, i, 0)),
            pl.BlockSpec((RB, D), lambda i: (i, 0)),
        ],
        out_specs=[
            pl.BlockSpec((RB, D), lambda i: (i, 0)),
            pl.BlockSpec((RB, D), lambda i: (i, 0)),
        ],
        out_shape=[
            jax.ShapeDtypeStruct((N, D), jnp.float32),
            jax.ShapeDtypeStruct((N, D), jnp.float32),
        ],
    )(degp, h)


def _layer(sp, g, h0, dis, w, theta):
    def body(sp_ref, g_ref, h0_ref, dis_ref, w_ref, o_ref):
        s = sp_ref[...]
        dis_b = dis_ref[...]
        hi = dis_b * (s[0] + s[1] + g_ref[...])
        sup = (1.0 - ALPHA) * hi + ALPHA * h0_ref[...]
        hn = jnp.maximum(
            theta * jnp.dot(sup, w_ref[...],
                            preferred_element_type=jnp.float32,
                            precision=lax.Precision.HIGHEST)
            + (1.0 - theta) * sup, 0.0)
        o_ref[...] = dis_b * hn

    return pl.pallas_call(
        body,
        grid=(N // RB,),
        in_specs=[
            pl.BlockSpec((NC, RB, D), lambda i: (0, i, 0)),
            pl.BlockSpec((RB, D), lambda i: (i, 0)),
            pl.BlockSpec((RB, D), lambda i: (i, 0)),
            pl.BlockSpec((RB, D), lambda i: (i, 0)),
            pl.BlockSpec((D, D), lambda i: (0, 0)),
        ],
        out_specs=pl.BlockSpec((RB, D), lambda i: (i, 0)),
        out_shape=jax.ShapeDtypeStruct((N, D), jnp.float32),
    )(sp, g, h0, dis, w)


def _final(sp, g1, h0, dis, w2, fc1_w, fc1_b):
    def body(sp_ref, g_ref, h0_ref, dis_ref, w_ref, fw_ref, fb_ref, o_ref):
        s = sp_ref[...]
        hi = dis_ref[...] * (s[0] + s[1] + g_ref[...])
        sup = (1.0 - ALPHA) * hi + ALPHA * h0_ref[...]
        h2 = jnp.maximum(
            THETA2 * jnp.dot(sup, w_ref[...],
                             preferred_element_type=jnp.float32,
                             precision=lax.Precision.HIGHEST)
            + (1.0 - THETA2) * sup, 0.0)
        o = jnp.dot(h2, fw_ref[...],
                    preferred_element_type=jnp.float32,
                    precision=lax.Precision.HIGHEST) + fb_ref[...]
        m = jnp.max(o, axis=1, keepdims=True)
        o_ref[...] = o - m - jnp.log(
            jnp.sum(jnp.exp(o - m), axis=1, keepdims=True))

    return pl.pallas_call(
        body,
        grid=(N // RB,),
        in_specs=[
            pl.BlockSpec((NC, RB, D), lambda i: (0, i, 0)),
            pl.BlockSpec((RB, D), lambda i: (i, 0)),
            pl.BlockSpec((RB, D), lambda i: (i, 0)),
            pl.BlockSpec((RB, D), lambda i: (i, 0)),
            pl.BlockSpec((D, D), lambda i: (0, 0)),
            pl.BlockSpec((D, D), lambda i: (0, 0)),
            pl.BlockSpec((1, D), lambda i: (0, 0)),
        ],
        out_specs=pl.BlockSpec((RB, D), lambda i: (i, 0)),
        out_shape=jax.ShapeDtypeStruct((N, D), jnp.float32),
    )(sp, g1, h0, dis, w2, fc1_w, fc1_b.reshape(1, D))


def kernel(x, edge_index, fc0_w, fc0_b, w1, w2, fc1_w, fc1_b):
    ar = jnp.arange(PADT, dtype=jnp.int32)
    pad_rc = jnp.stack([N + ar % (N_ACC - N), ar])
    eip = jnp.concatenate([edge_index, pad_rc], axis=1)
    rows = eip[0].reshape(NW, SNCH, SCHUNK)
    cols = eip[1].reshape(NW, SNCH, SCHUNK)
    colsd = eip[1].reshape(NW, NCH, CHUNK)
    zeros128 = jnp.zeros((TPW, D), jnp.float32)

    degp = _deg_partials(colsd)
    h0 = _fc0(x, fc0_w, fc0_b)
    dis, g0 = _prep(degp, h0)
    sp1 = _spmm_partials(g0, cols, rows, zeros128)
    g1 = _layer(sp1, g0, h0, dis, w1, THETA1)
    sp2 = _spmm_partials(g1, cols, rows, zeros128)
    return _final(sp2, g1, h0, dis, w2, fc1_w, fc1_b)


# speedup vs baseline: 1.1617x; 1.0519x over previous
import dataclasses
import functools
import math

import jax
import jax.numpy as jnp
import numpy as np
from jax import lax
from jax.experimental import pallas as pl
from jax.experimental.pallas import tpu as pltpu
from jax.experimental.pallas import tpu_sc as plsc

N = 10000
E = 320000
D = 128
NC = 2
NS = 16
NW = NC * NS
CHUNK = 64
NCH = 160
SCHUNK = 64
SNCH = 160
PNCH = 40
NPH = 4
NB = 4
E_PAD = NW * NCH * CHUNK
PADT = E_PAD - E
N_ACC = 10112
TPW = N_ACC // NS
N_DEG = 10240
TPD = N_DEG // NS
RB = 2000

_AR = np.arange(PADT, dtype=np.int32)
_PAD_RC = np.stack([N + _AR % (N_ACC - N), _AR])

ALPHA = 0.1
THETA1 = math.log(2.0)
THETA2 = math.log(1.5)

_MESH = plsc.VectorSubcoreMesh(core_axis_name="c", subcore_axis_name="s")

_CP = pltpu.CompilerParams()
if "needs_layout_passes" in pltpu.CompilerParams.__dataclass_fields__:
    _CP = dataclasses.replace(_CP, needs_layout_passes=False)


def _deg_partials(colh):
    @functools.partial(
        pl.kernel,
        out_type=jax.ShapeDtypeStruct((NC, N_DEG, D), jnp.float32),
        mesh=_MESH,
        compiler_params=_CP,
        scratch_types=[
            pltpu.VMEM((NCH, CHUNK), jnp.int32),
            pltpu.VMEM((N_DEG,), jnp.float32),
            pltpu.VMEM((NS, TPD), jnp.float32),
            pltpu.VMEM((TPD,), jnp.float32),
            pltpu.VMEM((TPD // 2, D), jnp.float32),
            pltpu.VMEM_SHARED((NS, N_DEG), jnp.float32),
        ],
    )
    def k(colh_hbm, out_hbm, colv, hist, redbuf, tot, outbuf, shist):
        c = lax.axis_index("c")
        s = lax.axis_index("s")
        w = c * NS + s
        pltpu.sync_copy(colh_hbm.at[w], colv)

        @pl.loop(0, N_DEG // 16)
        def _(i):
            hist[pl.ds(i * 16, 16)] = jnp.zeros((16,), jnp.float32)

        ones = jnp.ones((16,), jnp.float32)

        @pl.loop(0, NCH)
        def _(j):
            for k in range(CHUNK // 16):
                idx = colv[j, pl.ds(k * 16, 16)]
                plsc.addupdate_scatter(hist, [idx], ones)

        pltpu.sync_copy(hist, shist.at[s])
        plsc.subcore_barrier()
        for t in range(NS):
            pltpu.sync_copy(shist.at[t].at[pl.ds(s * TPD, TPD)],
                            redbuf.at[t])

        @pl.loop(0, TPD // 16)
        def _(i):
            v = redbuf[0, pl.ds(i * 16, 16)]
            for t in range(1, NS):
                v = v + redbuf[t, pl.ds(i * 16, 16)]
            tot[pl.ds(i * 16, 16)] = v

        for h in range(2):
            @pl.loop(0, TPD // 32)
            def _(i):
                v = tot[pl.ds(h * (TPD // 2) + i * 16, 16)]
                for k in range(16):
                    outbuf[i * 16 + k, pl.ds(0, 16)] = jnp.full((16,), v[k])

            pltpu.sync_copy(
                outbuf,
                out_hbm.at[c].at[pl.ds(s * TPD + h * (TPD // 2), TPD // 2)])

    return k(colh)


def _spmm_partials(g, colg, rows, zeros128):
    @functools.partial(
        pl.kernel,
        out_type=jax.ShapeDtypeStruct((NC, N_ACC, D), jnp.float32),
        mesh=_MESH,
        scratch_types=[
            pltpu.VMEM((PNCH, SCHUNK), jnp.int32),
            pltpu.VMEM((PNCH, SCHUNK), jnp.int32),
            pltpu.VMEM((SCHUNK, D), jnp.float32),
            pltpu.VMEM((SCHUNK, D), jnp.float32),
            pltpu.VMEM((SCHUNK, D), jnp.float32),
            pltpu.VMEM((SCHUNK, D), jnp.float32),
            pltpu.VMEM_SHARED((N_ACC, D), jnp.float32),
            pltpu.SemaphoreType.DMA,
            pltpu.SemaphoreType.DMA,
            pltpu.SemaphoreType.DMA,
            pltpu.SemaphoreType.DMA,
            pltpu.SemaphoreType.DMA,
            pltpu.SemaphoreType.DMA,
            pltpu.SemaphoreType.DMA,
            pltpu.SemaphoreType.DMA,
        ],
    )
    def k(g_hbm, colg_hbm, rows_hbm, zeros_hbm, out_hbm, colv, rowv,
          d0, d1, d2, d3, acc,
          gs0, gs1, gs2, gs3, ss0, ss1, ss2, ss3):
        c = lax.axis_index("c")
        s = lax.axis_index("s")
        w = c * NS + s
        pltpu.sync_copy(zeros_hbm, acc.at[pl.ds(s * TPW, TPW)])

        bufs = (d0, d1, d2, d3)
        gsems = (gs0, gs1, gs2, gs3)
        ssems = (ss0, ss1, ss2, ss3)

        for ph in range(NPH):
            pltpu.sync_copy(colg_hbm.at[w].at[pl.ds(ph * PNCH, PNCH)], colv)
            pltpu.sync_copy(rows_hbm.at[w].at[pl.ds(ph * PNCH, PNCH)], rowv)
            if ph == 0:
                plsc.subcore_barrier()
            for b in range(NB):
                pltpu.make_async_copy(
                    g_hbm.at[colv.at[b]], bufs[b], gsems[b]).start()

            @pl.loop(0, PNCH, step=NB)
            def _(j):
                for b in range(NB):
                    pltpu.make_async_copy(
                        g_hbm.at[colv.at[j + b]], bufs[b], gsems[b]).wait()
                    pltpu.make_async_copy(
                        bufs[b], acc.at[rowv.at[j + b]],
                        ssems[b]).start(add=True)
                for b in range(NB):
                    pltpu.make_async_copy(
                        bufs[b], acc.at[rowv.at[j + b]], ssems[b]).wait()

                    @pl.when(j + NB + b < PNCH)
                    def _():
                        pltpu.make_async_copy(
                            g_hbm.at[colv.at[j + NB + b]], bufs[b],
                            gsems[b]).start()

        plsc.subcore_barrier()
        pltpu.sync_copy(acc.at[pl.ds(s * TPW, TPW)],
                        out_hbm.at[c].at[pl.ds(s * TPW, TPW)])

    return k(g, colg, rows, zeros128)


def _fc0(x, fc0_w, fc0_b):
    def body(x_ref, w_ref, b_ref, o_ref):
        o_ref[...] = jnp.maximum(
            jnp.dot(x_ref[...], w_ref[...],
                    preferred_element_type=jnp.float32,
                    precision=lax.Precision.HIGHEST) + b_ref[...], 0.0)

    return pl.pallas_call(
        body,
        grid=(N // RB,),
        in_specs=[
            pl.BlockSpec((RB, D), lambda i: (i, 0)),
            pl.BlockSpec((D, D), lambda i: (0, 0)),
            pl.BlockSpec((1, D), lambda i: (0, 0)),
        ],
        out_specs=pl.BlockSpec((RB, D), lambda i: (i, 0)),
        out_shape=jax.ShapeDtypeStruct((N, D), jnp.float32),
    )(x, fc0_w, fc0_b.reshape(1, D))


def _prep(degp, h):
    def body(degp_ref, h_ref, dis_ref, g_ref):
        d = degp_ref[...]
        ids = (pl.program_id(0) * RB
               + lax.broadcasted_iota(jnp.int32, (RB, 1), 0))
        corr = jnp.where(ids < PADT, 1.0, 0.0)
        deg = d[0][:, :1] + d[1][:, :1] + 1.0 - corr
        dis = lax.rsqrt(deg)
        dis_b = jnp.broadcast_to(dis, (RB, D))
        dis_ref[...] = dis_b
        g_ref[...] = dis_b * h_ref[...]

    return pl.pallas_call(
        body,
        grid=(N // RB,),
        in_specs=[
            pl.BlockSpec((NC, RB, D), lambda i: (0, i, 0)),
            pl.BlockSpec((RB, D), lambda i: (i, 0)),
        ],
        out_specs=[
            pl.BlockSpec((RB, D), lambda i: (i, 0)),
            pl.BlockSpec((RB, D), lambda i: (i, 0)),
        ],
        out_shape=[
            jax.ShapeDtypeStruct((N, D), jnp.float32),
            jax.ShapeDtypeStruct((N, D), jnp.float32),
        ],
    )(degp, h)


def _layer(sp, g, h0, dis, w, theta):
    def body(sp_ref, g_ref, h0_ref, dis_ref, w_ref, o_ref):
        s = sp_ref[...]
        dis_b = dis_ref[...]
        hi = dis_b * (s[0] + s[1] + g_ref[...])
        sup = (1.0 - ALPHA) * hi + ALPHA * h0_ref[...]
        hn = jnp.maximum(
            theta * jnp.dot(sup, w_ref[...],
                            preferred_element_type=jnp.float32,
                            precision=lax.Precision.HIGHEST)
            + (1.0 - theta) * sup, 0.0)
        o_ref[...] = dis_b * hn

    return pl.pallas_call(
        body,
        grid=(N // RB,),
        in_specs=[
            pl.BlockSpec((NC, RB, D), lambda i: (0, i, 0)),
            pl.BlockSpec((RB, D), lambda i: (i, 0)),
            pl.BlockSpec((RB, D), lambda i: (i, 0)),
            pl.BlockSpec((RB, D), lambda i: (i, 0)),
            pl.BlockSpec((D, D), lambda i: (0, 0)),
        ],
        out_specs=pl.BlockSpec((RB, D), lambda i: (i, 0)),
        out_shape=jax.ShapeDtypeStruct((N, D), jnp.float32),
    )(sp, g, h0, dis, w)


def _final(sp, g1, h0, dis, w2, fc1_w, fc1_b):
    def body(sp_ref, g_ref, h0_ref, dis_ref, w_ref, fw_ref, fb_ref, o_ref):
        s = sp_ref[...]
        hi = dis_ref[...] * (s[0] + s[1] + g_ref[...])
        sup = (1.0 - ALPHA) * hi + ALPHA * h0_ref[...]
        h2 = jnp.maximum(
            THETA2 * jnp.dot(sup, w_ref[...],
                             preferred_element_type=jnp.float32,
                             precision=lax.Precision.HIGHEST)
            + (1.0 - THETA2) * sup, 0.0)
        o = jnp.dot(h2, fw_ref[...],
                    preferred_element_type=jnp.float32,
                    precision=lax.Precision.HIGHEST) + fb_ref[...]
        m = jnp.max(o, axis=1, keepdims=True)
        o_ref[...] = o - m - jnp.log(
            jnp.sum(jnp.exp(o - m), axis=1, keepdims=True))

    return pl.pallas_call(
        body,
        grid=(N // RB,),
        in_specs=[
            pl.BlockSpec((NC, RB, D), lambda i: (0, i, 0)),
            pl.BlockSpec((RB, D), lambda i: (i, 0)),
            pl.BlockSpec((RB, D), lambda i: (i, 0)),
            pl.BlockSpec((RB, D), lambda i: (i, 0)),
            pl.BlockSpec((D, D), lambda i: (0, 0)),
            pl.BlockSpec((D, D), lambda i: (0, 0)),
            pl.BlockSpec((1, D), lambda i: (0, 0)),
        ],
        out_specs=pl.BlockSpec((RB, D), lambda i: (i, 0)),
        out_shape=jax.ShapeDtypeStruct((N, D), jnp.float32),
    )(sp, g1, h0, dis, w2, fc1_w, fc1_b.reshape(1, D))


def kernel(x, edge_index, fc0_w, fc0_b, w1, w2, fc1_w, fc1_b):
    eip = jnp.concatenate([edge_index, _PAD_RC], axis=1)
    rows = eip[0].reshape(NW, SNCH, SCHUNK)
    cols = eip[1].reshape(NW, SNCH, SCHUNK)
    colsd = eip[1].reshape(NW, NCH, CHUNK)
    zeros128 = jnp.zeros((TPW, D), jnp.float32)

    degp = _deg_partials(colsd)
    h0 = _fc0(x, fc0_w, fc0_b)
    dis, g0 = _prep(degp, h0)
    sp1 = _spmm_partials(g0, cols, rows, zeros128)
    g1 = _layer(sp1, g0, h0, dis, w1, THETA1)
    sp2 = _spmm_partials(g1, cols, rows, zeros128)
    return _final(sp2, g1, h0, dis, w2, fc1_w, fc1_b)


# speedup vs baseline: 1.2287x; 1.0577x over previous
import dataclasses
import functools
import math

import jax
import jax.numpy as jnp
import numpy as np
from jax import lax
from jax.experimental import pallas as pl
from jax.experimental.pallas import tpu as pltpu
from jax.experimental.pallas import tpu_sc as plsc

N = 10000
E = 320000
D = 128
NC = 2
NS = 16
NW = NC * NS
CHUNK = 64
NCH = 160
SCHUNK = 64
SNCH = 160
PNCH = 40
NPH = 4
NB = 4
E_PAD = NW * NCH * CHUNK
PADT = E_PAD - E
N_ACC = 10112
TPW = N_ACC // NS
N_DEG = 10240
TPD = N_DEG // NS
RB = 2000

_AR = np.arange(PADT, dtype=np.int32)
_PAD_R3 = (N + _AR % (N_ACC - N)).reshape(PADT // 128, 128)
_PAD_C3 = _AR.reshape(PADT // 128, 128)

ALPHA = 0.1
THETA1 = math.log(2.0)
THETA2 = math.log(1.5)

_MESH = plsc.VectorSubcoreMesh(core_axis_name="c", subcore_axis_name="s")

_CP = pltpu.CompilerParams()
if "needs_layout_passes" in pltpu.CompilerParams.__dataclass_fields__:
    _CP = dataclasses.replace(_CP, needs_layout_passes=False)


def _deg_partials(colh):
    @functools.partial(
        pl.kernel,
        out_type=jax.ShapeDtypeStruct((NC, N_DEG, D), jnp.float32),
        mesh=_MESH,
        compiler_params=_CP,
        scratch_types=[
            pltpu.VMEM((NCH, CHUNK), jnp.int32),
            pltpu.VMEM((N_DEG,), jnp.float32),
            pltpu.VMEM((NS, TPD), jnp.float32),
            pltpu.VMEM((TPD,), jnp.float32),
            pltpu.VMEM((TPD // 2, D), jnp.float32),
            pltpu.VMEM_SHARED((NS, N_DEG), jnp.float32),
        ],
    )
    def k(colh_hbm, out_hbm, colv, hist, redbuf, tot, outbuf, shist):
        c = lax.axis_index("c")
        s = lax.axis_index("s")
        w = c * NS + s
        pltpu.sync_copy(colh_hbm.at[w], colv)

        @pl.loop(0, N_DEG // 16)
        def _(i):
            hist[pl.ds(i * 16, 16)] = jnp.zeros((16,), jnp.float32)

        ones = jnp.ones((16,), jnp.float32)

        @pl.loop(0, NCH)
        def _(j):
            for k in range(CHUNK // 16):
                idx = colv[j, pl.ds(k * 16, 16)]
                plsc.addupdate_scatter(hist, [idx], ones)

        pltpu.sync_copy(hist, shist.at[s])
        plsc.subcore_barrier()
        for t in range(NS):
            pltpu.sync_copy(shist.at[t].at[pl.ds(s * TPD, TPD)],
                            redbuf.at[t])

        @pl.loop(0, TPD // 16)
        def _(i):
            v = redbuf[0, pl.ds(i * 16, 16)]
            for t in range(1, NS):
                v = v + redbuf[t, pl.ds(i * 16, 16)]
            tot[pl.ds(i * 16, 16)] = v

        for h in range(2):
            @pl.loop(0, TPD // 32)
            def _(i):
                v = tot[pl.ds(h * (TPD // 2) + i * 16, 16)]
                for k in range(16):
                    outbuf[i * 16 + k, pl.ds(0, 16)] = jnp.full((16,), v[k])

            pltpu.sync_copy(
                outbuf,
                out_hbm.at[c].at[pl.ds(s * TPD + h * (TPD // 2), TPD // 2)])

    return k(colh)


def _spmm_partials(g, colg, rows, zeros128):
    @functools.partial(
        pl.kernel,
        out_type=jax.ShapeDtypeStruct((NC, N_ACC, D), jnp.float32),
        mesh=_MESH,
        scratch_types=[
            pltpu.VMEM((PNCH, SCHUNK), jnp.int32),
            pltpu.VMEM((PNCH, SCHUNK), jnp.int32),
            pltpu.VMEM((SCHUNK, D), jnp.float32),
            pltpu.VMEM((SCHUNK, D), jnp.float32),
            pltpu.VMEM((SCHUNK, D), jnp.float32),
            pltpu.VMEM((SCHUNK, D), jnp.float32),
            pltpu.VMEM_SHARED((N_ACC, D), jnp.float32),
            pltpu.SemaphoreType.DMA,
            pltpu.SemaphoreType.DMA,
            pltpu.SemaphoreType.DMA,
            pltpu.SemaphoreType.DMA,
            pltpu.SemaphoreType.DMA,
            pltpu.SemaphoreType.DMA,
            pltpu.SemaphoreType.DMA,
            pltpu.SemaphoreType.DMA,
        ],
    )
    def k(g_hbm, colg_hbm, rows_hbm, zeros_hbm, out_hbm, colv, rowv,
          d0, d1, d2, d3, acc,
          gs0, gs1, gs2, gs3, ss0, ss1, ss2, ss3):
        c = lax.axis_index("c")
        s = lax.axis_index("s")
        w = c * NS + s
        pltpu.sync_copy(zeros_hbm, acc.at[pl.ds(s * TPW, TPW)])

        bufs = (d0, d1, d2, d3)
        gsems = (gs0, gs1, gs2, gs3)
        ssems = (ss0, ss1, ss2, ss3)

        for ph in range(NPH):
            pltpu.sync_copy(colg_hbm.at[w].at[pl.ds(ph * PNCH, PNCH)], colv)
            pltpu.sync_copy(rows_hbm.at[w].at[pl.ds(ph * PNCH, PNCH)], rowv)
            if ph == 0:
                plsc.subcore_barrier()
            for b in range(NB):
                pltpu.make_async_copy(
                    g_hbm.at[colv.at[b]], bufs[b], gsems[b]).start()

            @pl.loop(0, PNCH, step=NB)
            def _(j):
                for b in range(NB):
                    pltpu.make_async_copy(
                        g_hbm.at[colv.at[j + b]], bufs[b], gsems[b]).wait()
                    pltpu.make_async_copy(
                        bufs[b], acc.at[rowv.at[j + b]],
                        ssems[b]).start(add=True)
                for b in range(NB):
                    pltpu.make_async_copy(
                        bufs[b], acc.at[rowv.at[j + b]], ssems[b]).wait()

                    @pl.when(j + NB + b < PNCH)
                    def _():
                        pltpu.make_async_copy(
                            g_hbm.at[colv.at[j + NB + b]], bufs[b],
                            gsems[b]).start()

        plsc.subcore_barrier()
        pltpu.sync_copy(acc.at[pl.ds(s * TPW, TPW)],
                        out_hbm.at[c].at[pl.ds(s * TPW, TPW)])

    return k(g, colg, rows, zeros128)


def _fc0(x, fc0_w, fc0_b):
    def body(x_ref, w_ref, b_ref, o_ref):
        o_ref[...] = jnp.maximum(
            jnp.dot(x_ref[...], w_ref[...],
                    preferred_element_type=jnp.float32) + b_ref[...], 0.0)

    return pl.pallas_call(
        body,
        grid=(N // RB,),
        in_specs=[
            pl.BlockSpec((RB, D), lambda i: (i, 0)),
            pl.BlockSpec((D, D), lambda i: (0, 0)),
            pl.BlockSpec((1, D), lambda i: (0, 0)),
        ],
        out_specs=pl.BlockSpec((RB, D), lambda i: (i, 0)),
        out_shape=jax.ShapeDtypeStruct((N, D), jnp.float32),
    )(x, fc0_w, fc0_b.reshape(1, D))


def _prep(degp, h):
    def body(degp_ref, h_ref, dis_ref, g_ref):
        d = degp_ref[...]
        ids = (pl.program_id(0) * RB
               + lax.broadcasted_iota(jnp.int32, (RB, 1), 0))
        corr = jnp.where(ids < PADT, 1.0, 0.0)
        deg = d[0][:, :1] + d[1][:, :1] + 1.0 - corr
        dis = lax.rsqrt(deg)
        dis_b = jnp.broadcast_to(dis, (RB, D))
        dis_ref[...] = dis_b
        g_ref[...] = dis_b * h_ref[...]

    return pl.pallas_call(
        body,
        grid=(N // RB,),
        in_specs=[
            pl.BlockSpec((NC, RB, D), lambda i: (0, i, 0)),
            pl.BlockSpec((RB, D), lambda i: (i, 0)),
        ],
        out_specs=[
            pl.BlockSpec((RB, D), lambda i: (i, 0)),
            pl.BlockSpec((RB, D), lambda i: (i, 0)),
        ],
        out_shape=[
            jax.ShapeDtypeStruct((N, D), jnp.float32),
            jax.ShapeDtypeStruct((N, D), jnp.float32),
        ],
    )(degp, h)


def _layer(sp, g, h0, dis, w, theta):
    def body(sp_ref, g_ref, h0_ref, dis_ref, w_ref, o_ref):
        s = sp_ref[...]
        dis_b = dis_ref[...]
        hi = dis_b * (s[0] + s[1] + g_ref[...])
        sup = (1.0 - ALPHA) * hi + ALPHA * h0_ref[...]
        hn = jnp.maximum(
            theta * jnp.dot(sup, w_ref[...],
                            preferred_element_type=jnp.float32)
            + (1.0 - theta) * sup, 0.0)
        o_ref[...] = dis_b * hn

    return pl.pallas_call(
        body,
        grid=(N // RB,),
        in_specs=[
            pl.BlockSpec((NC, RB, D), lambda i: (0, i, 0)),
            pl.BlockSpec((RB, D), lambda i: (i, 0)),
            pl.BlockSpec((RB, D), lambda i: (i, 0)),
            pl.BlockSpec((RB, D), lambda i: (i, 0)),
            pl.BlockSpec((D, D), lambda i: (0, 0)),
        ],
        out_specs=pl.BlockSpec((RB, D), lambda i: (i, 0)),
        out_shape=jax.ShapeDtypeStruct((N, D), jnp.float32),
    )(sp, g, h0, dis, w)


def _final(sp, g1, h0, dis, w2, fc1_w, fc1_b):
    def body(sp_ref, g_ref, h0_ref, dis_ref, w_ref, fw_ref, fb_ref, o_ref):
        s = sp_ref[...]
        hi = dis_ref[...] * (s[0] + s[1] + g_ref[...])
        sup = (1.0 - ALPHA) * hi + ALPHA * h0_ref[...]
        h2 = jnp.maximum(
            THETA2 * jnp.dot(sup, w_ref[...],
                             preferred_element_type=jnp.float32)
            + (1.0 - THETA2) * sup, 0.0)
        o = jnp.dot(h2, fw_ref[...],
                    preferred_element_type=jnp.float32) + fb_ref[...]
        m = jnp.max(o, axis=1, keepdims=True)
        o_ref[...] = o - m - jnp.log(
            jnp.sum(jnp.exp(o - m), axis=1, keepdims=True))

    return pl.pallas_call(
        body,
        grid=(N // RB,),
        in_specs=[
            pl.BlockSpec((NC, RB, D), lambda i: (0, i, 0)),
            pl.BlockSpec((RB, D), lambda i: (i, 0)),
            pl.BlockSpec((RB, D), lambda i: (i, 0)),
            pl.BlockSpec((RB, D), lambda i: (i, 0)),
            pl.BlockSpec((D, D), lambda i: (0, 0)),
            pl.BlockSpec((D, D), lambda i: (0, 0)),
            pl.BlockSpec((1, D), lambda i: (0, 0)),
        ],
        out_specs=pl.BlockSpec((RB, D), lambda i: (i, 0)),
        out_shape=jax.ShapeDtypeStruct((N, D), jnp.float32),
    )(sp, g1, h0, dis, w2, fc1_w, fc1_b.reshape(1, D))


def kernel(x, edge_index, fc0_w, fc0_b, w1, w2, fc1_w, fc1_b):
    ei128 = edge_index.reshape(2, E // 128, 128)
    rows = jnp.concatenate([ei128[0], _PAD_R3], axis=0)
    cols = jnp.concatenate([ei128[1], _PAD_C3], axis=0)
    rows = rows.reshape(NW, SNCH, SCHUNK)
    colsd = cols.reshape(NW, NCH, CHUNK)
    cols = cols.reshape(NW, SNCH, SCHUNK)
    zeros128 = jnp.zeros((TPW, D), jnp.float32)

    degp = _deg_partials(colsd)
    h0 = _fc0(x, fc0_w, fc0_b)
    dis, g0 = _prep(degp, h0)
    sp1 = _spmm_partials(g0, cols, rows, zeros128)
    g1 = _layer(sp1, g0, h0, dis, w1, THETA1)
    sp2 = _spmm_partials(g1, cols, rows, zeros128)
    return _final(sp2, g1, h0, dis, w2, fc1_w, fc1_b)
